# Initial kernel scaffold; baseline (speedup 1.0000x reference)
import jax
import jax.numpy as jnp
from jax.experimental import pallas as pl


def kernel(x, edge_idx, w_pair, w_each):
    raise NotImplementedError("write your pallas kernel here")


# speedup vs baseline: 5.4945x; 5.4945x over previous
import functools

import jax
import jax.numpy as jnp
from jax import lax
from jax.experimental import pallas as pl
from jax.experimental.pallas import tpu as pltpu
from jax.experimental.pallas import tpu_sc as plsc

N = 10000
D = 128
E = 320000
AW = 144
NC, NS = 2, 16
NW = NC * NS
EPW = E // NW
K = 80
NCHUNK = EPW // K
RPT = 640
R2 = 80
NV = AW // 16


def _prep_body(x_ref, wp_ref, we_ref, a_ref, b_ref, each_ref):
    i = pl.program_id(0)
    xb = x_ref[...]
    wp = wp_ref[...]
    we = we_ref[...]
    z2 = xb * wp * (-2.0)
    q = jnp.sum(xb * xb * wp, axis=1, keepdims=True)
    ones = jnp.ones_like(q)
    pad = jnp.zeros((xb.shape[0], AW - D - 2), xb.dtype)
    a_ref[...] = jnp.concatenate([z2, ones, q, pad], axis=1)
    b_ref[...] = jnp.concatenate([xb, q, ones, pad], axis=1)

    @pl.when(i == 0)
    def _():
        each_ref[...] = jnp.zeros_like(each_ref)

    each_ref[...] = each_ref[...] + jnp.sum(xb * we).reshape(1, 1)


def _prep(xf, wp_row, we_row):
    rows = 1000
    grid = N // rows
    return pl.pallas_call(
        _prep_body,
        grid=(grid,),
        in_specs=[
            pl.BlockSpec((rows, D), lambda i: (i, 0)),
            pl.BlockSpec((1, D), lambda i: (0, 0)),
            pl.BlockSpec((1, D), lambda i: (0, 0)),
        ],
        out_specs=[
            pl.BlockSpec((rows, AW), lambda i: (i, 0)),
            pl.BlockSpec((rows, AW), lambda i: (i, 0)),
            pl.BlockSpec((1, 1), lambda i: (0, 0)),
        ],
        out_shape=[
            jax.ShapeDtypeStruct((N, AW), jnp.float32),
            jax.ShapeDtypeStruct((N, AW), jnp.float32),
            jax.ShapeDtypeStruct((1, 1), jnp.float32),
        ],
    )(xf, wp_row, we_row)


def _sc_edge_body(a_hbm, b_hbm, edge_hbm, out_hbm,
                  S, idx2, abuf, sbuf, bbuf, obuf, sem):
    c = lax.axis_index("c")
    s = lax.axis_index("s")
    wid = c * NS + s
    row_base = s * RPT
    n_sub = lax.select(s == NS - 1, 5, 8)

    zv = jnp.zeros((16,), jnp.float32)

    def zrow(r, carry):
        for v in range(NV):
            sbuf[r, pl.ds(v * 16, 16)] = zv
        return carry

    lax.fori_loop(0, R2, zrow, 0)

    def zcopy(t, carry):
        pltpu.sync_copy(sbuf, S.at[pl.ds(row_base + t * R2, R2)])
        return carry

    lax.fori_loop(0, n_sub, zcopy, 0)
    plsc.subcore_barrier()

    def step(j, carry):
        pltpu.sync_copy(edge_hbm.at[wid, j], idx2)
        pltpu.async_copy(a_hbm.at[idx2.at[0]], abuf, sem).wait()
        pltpu.sync_copy(abuf, S.at[idx2.at[1]], add=True)
        return carry

    lax.fori_loop(0, NCHUNK, step, 0)
    plsc.subcore_barrier()

    def red_chunk(t, acc):
        base = row_base + t * R2
        pltpu.sync_copy(S.at[pl.ds(base, R2)], sbuf)
        pltpu.sync_copy(b_hbm.at[pl.ds(base, R2)], bbuf)

        def rrow(r, acc):
            for v in range(NV):
                acc = acc + sbuf[r, pl.ds(v * 16, 16)] * bbuf[r, pl.ds(v * 16, 16)]
            return acc

        return lax.fori_loop(0, R2, rrow, acc)

    acc = lax.fori_loop(0, n_sub, red_chunk, jnp.zeros((16,), jnp.float32))
    obuf[...] = acc
    pltpu.sync_copy(obuf, out_hbm.at[pl.ds(wid * 16, 16)])


@functools.cache
def _build_sc_kernel():
    mesh = plsc.VectorSubcoreMesh(
        core_axis_name="c", subcore_axis_name="s", num_cores=NC, num_subcores=NS
    )
    return pl.kernel(
        _sc_edge_body,
        out_type=jax.ShapeDtypeStruct((NW * 16,), jnp.float32),
        mesh=mesh,
        compiler_params=pltpu.CompilerParams(use_tc_tiling_on_sc=False),
        scratch_types=[
            pltpu.VMEM_SHARED((N, AW), jnp.float32),
            pltpu.VMEM((2, K), jnp.int32),
            pltpu.VMEM((K, AW), jnp.float32),
            pltpu.VMEM((R2, AW), jnp.float32),
            pltpu.VMEM((R2, AW), jnp.float32),
            pltpu.VMEM((16,), jnp.float32),
            pltpu.SemaphoreType.DMA,
        ],
    )


def kernel(x, edge_idx, w_pair, w_each):
    xf = x.reshape(N, D)
    a_tab, b_tab, each = _prep(xf, w_pair.reshape(1, D), w_each.reshape(1, D))
    edges = jnp.stack(
        [edge_idx[0].reshape(NW, NCHUNK, K), edge_idx[1].reshape(NW, NCHUNK, K)],
        axis=2,
    )
    parts = _build_sc_kernel()(a_tab, b_tab, edges)
    return (jnp.sum(parts) + each[0, 0]).reshape(1)
